# Initial kernel scaffold; baseline (speedup 1.0000x reference)
import jax
import jax.numpy as jnp
from jax.experimental import pallas as pl


def kernel(x, edge_index, batch, W1, a1_src, a1_dst, b1, W2, a2_src, a2_dst, b2, W3, a3_src, a3_dst, b3, W4, a4_src, a4_dst, b4):
    raise NotImplementedError("write your pallas kernel here")


# speedup vs baseline: 35.0307x; 35.0307x over previous
import jax
import jax.numpy as jnp
from jax import lax
from jax.experimental import pallas as pl
from jax.experimental.pallas import tpu as pltpu
from jax.experimental.pallas import tpu_sc as plsc

N = 10000
E = 320000
IN_DIM = 128
HID = 32
G = 256

NC = 2
NS = 16
NP = 10240
NPT = NP // NS
TRASH = 10200
B = 128
KB = 80
EPAD = NC * NS * KB * B
PCHUNK = 2000
PNB = N // PCHUNK


def _sc_edge_body(ta_hbm, tb_hbm, e_hbm, f_hbm, me_hbm, src_hbm, dst_hbm,
                  za_out, zb_out, aa_out, ab_out,
                  ta_sh, tb_sh, e_sh, f_sh, za_sh, zb_sh, aa_sh, ab_sh,
                  src_v, dst_v,
                  eg0, fg0, ra0, rb0, va0, vb0, ia0, ib0,
                  eg1, fg1, ra1, rb1, va1, vb1, ia1, ib1,
                  me_v,
                  gsem0, gsem1, ssem0, ssem1):
    cid = lax.axis_index("c")
    tid = lax.axis_index("s")
    nbase = tid * NPT

    pltpu.sync_copy(ta_hbm.at[pl.ds(nbase, NPT)], ta_sh.at[pl.ds(nbase, NPT)])
    pltpu.sync_copy(tb_hbm.at[pl.ds(nbase, NPT)], tb_sh.at[pl.ds(nbase, NPT)])
    pltpu.sync_copy(e_hbm.at[pl.ds(nbase, NPT)], e_sh.at[pl.ds(nbase, NPT)])
    pltpu.sync_copy(f_hbm.at[pl.ds(nbase, NPT)], f_sh.at[pl.ds(nbase, NPT)])
    pltpu.sync_copy(me_hbm, me_v)

    z16 = jnp.zeros((16,), jnp.float32)

    def zrow(r, c):
        ra0[r, pl.ds(0, 16)] = z16
        ra0[r, pl.ds(16, 16)] = z16
        return c
    lax.fori_loop(0, B, zrow, 0, unroll=4)

    def zrow1(k, c):
        va0[pl.ds(k * 16, 16)] = z16
        return c
    lax.fori_loop(0, B // 16, zrow1, 0, unroll=4)
    for q in range(NPT // B):
        sl = pl.ds(nbase + q * B, B)
        pltpu.sync_copy(ra0, aa_sh.at[sl])
        pltpu.sync_copy(ra0, ab_sh.at[sl])
        pltpu.sync_copy(va0, za_sh.at[sl])
        pltpu.sync_copy(va0, zb_sh.at[sl])

    g = cid * NS + tid
    pltpu.sync_copy(src_hbm.at[g], src_v)
    pltpu.sync_copy(dst_hbm.at[g], dst_v)

    plsc.subcore_barrier()
    mev = me_v[...]
    trash16 = jnp.full((16,), TRASH, jnp.int32)

    def fire_gathers(j, eg, fg, ra, rb, gsem):
        sidx = src_v.at[j]
        didx = dst_v.at[j]
        pltpu.async_copy(e_sh.at[sidx], eg, gsem)
        pltpu.async_copy(f_sh.at[didx], fg, gsem)
        pltpu.async_copy(ta_sh.at[sidx], ra, gsem)
        pltpu.async_copy(tb_sh.at[sidx], rb, gsem)

    def wait_gathers(j, eg, fg, ra, rb, gsem):
        sidx = src_v.at[j]
        didx = dst_v.at[j]
        pltpu.make_async_copy(e_sh.at[sidx], eg, gsem).wait()
        pltpu.make_async_copy(f_sh.at[didx], fg, gsem).wait()
        pltpu.make_async_copy(ta_sh.at[sidx], ra, gsem).wait()
        pltpu.make_async_copy(tb_sh.at[sidx], rb, gsem).wait()

    def compute(j, eg, fg, va, vb, ia, ib):
        for k in range(B // 16):
            sl = pl.ds(k * 16, 16)
            ev = eg[sl]
            fv = fg[sl]
            pos = (ev + fv) > 0.0
            dv = dst_v[j, sl]
            ia[sl] = jnp.where(pos, dv, trash16)
            ib[sl] = jnp.where(pos, trash16, dv)
            em = ev - mev
            va[sl] = jnp.exp(em)
            vb[sl] = jnp.exp(0.2 * em)

    def fire_scatters(va, vb, ia, ib, ra, rb, ssem):
        pltpu.async_copy(va, za_sh.at[ia], ssem, add=True)
        pltpu.async_copy(ra, aa_sh.at[ia], ssem, add=True)
        pltpu.async_copy(vb, zb_sh.at[ib], ssem, add=True)
        pltpu.async_copy(rb, ab_sh.at[ib], ssem, add=True)

    def wait_scatters(va, vb, ia, ib, ra, rb, ssem):
        pltpu.make_async_copy(va, za_sh.at[ia], ssem).wait()
        pltpu.make_async_copy(ra, aa_sh.at[ia], ssem).wait()
        pltpu.make_async_copy(vb, zb_sh.at[ib], ssem).wait()
        pltpu.make_async_copy(rb, ab_sh.at[ib], ssem).wait()

    fire_gathers(0, eg0, fg0, ra0, rb0, gsem0)

    def pair(i, c):
        b0 = 2 * i
        b1 = b0 + 1
        @pl.when(i > 0)
        def _():
            wait_scatters(va1, vb1, ia1, ib1, ra1, rb1, ssem1)
        fire_gathers(b1, eg1, fg1, ra1, rb1, gsem1)
        wait_gathers(b0, eg0, fg0, ra0, rb0, gsem0)
        compute(b0, eg0, fg0, va0, vb0, ia0, ib0)
        fire_scatters(va0, vb0, ia0, ib0, ra0, rb0, ssem0)
        wait_gathers(b1, eg1, fg1, ra1, rb1, gsem1)
        compute(b1, eg1, fg1, va1, vb1, ia1, ib1)
        wait_scatters(va0, vb0, ia0, ib0, ra0, rb0, ssem0)

        @pl.when(b1 + 1 < KB)
        def _():
            fire_gathers(b1 + 1, eg0, fg0, ra0, rb0, gsem0)
        fire_scatters(va1, vb1, ia1, ib1, ra1, rb1, ssem1)
        return c
    lax.fori_loop(0, KB // 2, pair, 0)
    wait_scatters(va1, vb1, ia1, ib1, ra1, rb1, ssem1)

    plsc.subcore_barrier()
    sl = pl.ds(nbase, NPT)
    pltpu.sync_copy(za_sh.at[sl], za_out.at[cid, sl])
    pltpu.sync_copy(zb_sh.at[sl], zb_out.at[cid, sl])
    pltpu.sync_copy(aa_sh.at[sl], aa_out.at[cid, sl])
    pltpu.sync_copy(ab_sh.at[sl], ab_out.at[cid, sl])


_sc_edge = pl.kernel(
    _sc_edge_body,
    out_type=(jax.ShapeDtypeStruct((NC, NP), jnp.float32),
              jax.ShapeDtypeStruct((NC, NP), jnp.float32),
              jax.ShapeDtypeStruct((NC, NP, HID), jnp.float32),
              jax.ShapeDtypeStruct((NC, NP, HID), jnp.float32)),
    mesh=plsc.VectorSubcoreMesh(core_axis_name="c", subcore_axis_name="s"),
    compiler_params=pltpu.CompilerParams(needs_layout_passes=False,
                                         use_tc_tiling_on_sc=False),
    scratch_types=[
        pltpu.VMEM_SHARED((NP, HID), jnp.float32),
        pltpu.VMEM_SHARED((NP, HID), jnp.float32),
        pltpu.VMEM_SHARED((NP,), jnp.float32),
        pltpu.VMEM_SHARED((NP,), jnp.float32),
        pltpu.VMEM_SHARED((NP,), jnp.float32),
        pltpu.VMEM_SHARED((NP,), jnp.float32),
        pltpu.VMEM_SHARED((NP, HID), jnp.float32),
        pltpu.VMEM_SHARED((NP, HID), jnp.float32),
        pltpu.VMEM((KB, B), jnp.int32),
        pltpu.VMEM((KB, B), jnp.int32),
        pltpu.VMEM((B,), jnp.float32),
        pltpu.VMEM((B,), jnp.float32),
        pltpu.VMEM((B, HID), jnp.float32),
        pltpu.VMEM((B, HID), jnp.float32),
        pltpu.VMEM((B,), jnp.float32),
        pltpu.VMEM((B,), jnp.float32),
        pltpu.VMEM((B,), jnp.int32),
        pltpu.VMEM((B,), jnp.int32),
        pltpu.VMEM((B,), jnp.float32),
        pltpu.VMEM((B,), jnp.float32),
        pltpu.VMEM((B, HID), jnp.float32),
        pltpu.VMEM((B, HID), jnp.float32),
        pltpu.VMEM((B,), jnp.float32),
        pltpu.VMEM((B,), jnp.float32),
        pltpu.VMEM((B,), jnp.int32),
        pltpu.VMEM((B,), jnp.int32),
        pltpu.VMEM((16,), jnp.float32),
        pltpu.SemaphoreType.DMA,
        pltpu.SemaphoreType.DMA,
        pltpu.SemaphoreType.DMA,
        pltpu.SemaphoreType.DMA,
    ],
)


def _proj(h, as_ref, ad_ref):
    e = jnp.sum(h * as_ref[...][None, :], axis=1)
    f = jnp.sum(h * ad_ref[...][None, :], axis=1)
    return e, f


def _emit_layer(h, as_ref, ad_ref, h_ref, ta_ref, tb_ref, e_ref, f_ref,
                me_ref, mf_ref):
    h_ref[...] = h
    e, f = _proj(h, as_ref, ad_ref)
    e_ref[...] = e
    f_ref[...] = f
    me = jnp.max(e)
    mf = jnp.max(f)
    me_ref[0, 0] = me
    mf_ref[0, 0] = mf
    ea = jnp.exp(e - me)
    eb = jnp.exp(0.2 * (e - me))
    ta_ref[...] = ea[:, None] * h
    tb_ref[...] = eb[:, None] * h


def _tc_first_body(x_ref, w_ref, as_ref, ad_ref,
                   h_ref, ta_ref, tb_ref, e_ref, f_ref, me_ref, mf_ref):
    h = jnp.dot(x_ref[...], w_ref[...], preferred_element_type=jnp.float32)
    _emit_layer(h, as_ref, ad_ref, h_ref, ta_ref, tb_ref, e_ref, f_ref,
                me_ref, mf_ref)


_LAYER_OUT = (jax.ShapeDtypeStruct((N, HID), jnp.float32),
              jax.ShapeDtypeStruct((N, HID), jnp.float32),
              jax.ShapeDtypeStruct((N, HID), jnp.float32),
              jax.ShapeDtypeStruct((N,), jnp.float32),
              jax.ShapeDtypeStruct((N,), jnp.float32),
              jax.ShapeDtypeStruct((1, 1), jnp.float32),
              jax.ShapeDtypeStruct((1, 1), jnp.float32))

_LAYER_OUT_SPECS = (pl.BlockSpec(memory_space=pltpu.VMEM),
                    pl.BlockSpec(memory_space=pltpu.VMEM),
                    pl.BlockSpec(memory_space=pltpu.VMEM),
                    pl.BlockSpec(memory_space=pltpu.VMEM),
                    pl.BlockSpec(memory_space=pltpu.VMEM),
                    pl.BlockSpec(memory_space=pltpu.SMEM),
                    pl.BlockSpec(memory_space=pltpu.SMEM))

_TC_VMEM = pltpu.CompilerParams(vmem_limit_bytes=110 * 1024 * 1024)

_tc_first = pl.pallas_call(
    _tc_first_body,
    out_shape=_LAYER_OUT,
    out_specs=_LAYER_OUT_SPECS,
    compiler_params=_TC_VMEM,
)


def _norm_nodes(za, zb, aa, ab, e_p, f_p, me, mf, hp, bp):
    M = me + mf
    fd = f_p - mf
    wA = jnp.exp(fd)
    wB = jnp.exp(0.2 * fd - 0.8 * M)
    s = e_p + f_p
    psl = jnp.exp(jnp.maximum(s, 0.2 * s) - M)
    z = wA * za + wB * zb + psl + 1e-16
    num = wA[:, None] * aa + wB[:, None] * ab + psl[:, None] * hp
    return num / z[:, None] + bp[None, :]


def _tc_mid_body(za_ref, zb_ref, aa_ref, ab_ref, ep_ref, fp_ref,
                 me_ref, mf_ref, hp_ref, bp_ref, w_ref, as_ref, ad_ref,
                 h_ref, ta_ref, tb_ref, e_ref, f_ref, meo_ref, mfo_ref):
    xin = _norm_nodes(za_ref[0] + za_ref[1], zb_ref[0] + zb_ref[1],
                      aa_ref[0] + aa_ref[1], ab_ref[0] + ab_ref[1],
                      ep_ref[...], fp_ref[...], me_ref[0, 0], mf_ref[0, 0],
                      hp_ref[...], bp_ref[...])
    h = jnp.dot(xin, w_ref[...], preferred_element_type=jnp.float32)
    _emit_layer(h, as_ref, ad_ref, h_ref, ta_ref, tb_ref, e_ref, f_ref,
                meo_ref, mfo_ref)


_tc_mid = pl.pallas_call(
    _tc_mid_body,
    out_shape=_LAYER_OUT,
    out_specs=_LAYER_OUT_SPECS,
    in_specs=[pl.BlockSpec(memory_space=pltpu.VMEM),
              pl.BlockSpec(memory_space=pltpu.VMEM),
              pl.BlockSpec(memory_space=pltpu.VMEM),
              pl.BlockSpec(memory_space=pltpu.VMEM),
              pl.BlockSpec(memory_space=pltpu.VMEM),
              pl.BlockSpec(memory_space=pltpu.VMEM),
              pl.BlockSpec(memory_space=pltpu.SMEM),
              pl.BlockSpec(memory_space=pltpu.SMEM),
              pl.BlockSpec(memory_space=pltpu.VMEM),
              pl.BlockSpec(memory_space=pltpu.VMEM),
              pl.BlockSpec(memory_space=pltpu.VMEM),
              pl.BlockSpec(memory_space=pltpu.VMEM),
              pl.BlockSpec(memory_space=pltpu.VMEM)],
    compiler_params=_TC_VMEM,
)


def _tc_pool_body(za_ref, zb_ref, aa_ref, ab_ref, ep_ref, fp_ref,
                  me_ref, mf_ref, hp_ref, bp_ref, batch_ref,
                  o_ref, sums_s, cnt_s):
    i = pl.program_id(0)
    xin = _norm_nodes(za_ref[0, 0, 0, :] + za_ref[1, 0, 0, :],
                      zb_ref[0, 0, 0, :] + zb_ref[1, 0, 0, :],
                      aa_ref[0] + aa_ref[1], ab_ref[0] + ab_ref[1],
                      ep_ref[0, 0, :], fp_ref[0, 0, :],
                      me_ref[0, 0], mf_ref[0, 0],
                      hp_ref[...], bp_ref[...])
    bt = batch_ref[0, 0, :]
    oh = (lax.broadcasted_iota(jnp.int32, (G, PCHUNK), 0)
          == bt[None, :]).astype(jnp.float32)

    @pl.when(i == 0)
    def _():
        sums_s[...] = jnp.zeros_like(sums_s)
        cnt_s[...] = jnp.zeros_like(cnt_s)

    sums_s[...] += jnp.dot(oh, xin, preferred_element_type=jnp.float32)
    cnt_s[...] += jnp.sum(oh, axis=1)

    @pl.when(i == pl.num_programs(0) - 1)
    def _():
        o_ref[...] = sums_s[...] / jnp.maximum(cnt_s[...], 1.0)[:, None]


_tc_pool = pl.pallas_call(
    _tc_pool_body,
    grid=(PNB,),
    in_specs=[
        pl.BlockSpec((NC, 1, 1, PCHUNK), lambda i: (0, i, 0, 0)),
        pl.BlockSpec((NC, 1, 1, PCHUNK), lambda i: (0, i, 0, 0)),
        pl.BlockSpec((NC, PCHUNK, HID), lambda i: (0, i, 0)),
        pl.BlockSpec((NC, PCHUNK, HID), lambda i: (0, i, 0)),
        pl.BlockSpec((1, 1, PCHUNK), lambda i: (i, 0, 0)),
        pl.BlockSpec((1, 1, PCHUNK), lambda i: (i, 0, 0)),
        pl.BlockSpec((1, 1), lambda i: (0, 0), memory_space=pltpu.SMEM),
        pl.BlockSpec((1, 1), lambda i: (0, 0), memory_space=pltpu.SMEM),
        pl.BlockSpec((PCHUNK, HID), lambda i: (i, 0)),
        pl.BlockSpec((HID,), lambda i: (0,)),
        pl.BlockSpec((1, 1, PCHUNK), lambda i: (i, 0, 0)),
    ],
    out_specs=pl.BlockSpec((G, HID), lambda i: (0, 0)),
    out_shape=jax.ShapeDtypeStruct((G, HID), jnp.float32),
    scratch_shapes=[pltpu.VMEM((G, HID), jnp.float32),
                    pltpu.VMEM((G,), jnp.float32)],
    compiler_params=_TC_VMEM,
)


def kernel(x, edge_index, batch,
           W1, a1_src, a1_dst, b1,
           W2, a2_src, a2_dst, b2,
           W3, a3_src, a3_dst, b3,
           W4, a4_src, a4_dst, b4):
    src = edge_index[0].astype(jnp.int32)
    dst = edge_index[1].astype(jnp.int32)
    pad_e = jnp.full((EPAD - E,), TRASH, jnp.int32)
    srcp = jnp.concatenate([src, pad_e]).reshape(NC * NS, KB, B)
    dstp = jnp.concatenate([dst, pad_e]).reshape(NC * NS, KB, B)
    zpadN = jnp.zeros((NP - N,), jnp.float32)
    zpadH = jnp.zeros((NP - N, HID), jnp.float32)

    params = [(W1, a1_src, a1_dst, b1), (W2, a2_src, a2_dst, b2),
              (W3, a3_src, a3_dst, b3), (W4, a4_src, a4_dst, b4)]

    h = ta = tb = e = f = me = mf = None
    za = zb = aa = ab = None
    for k in range(4):
        W, asrc, adst, _ = params[k]
        if k == 0:
            h, ta, tb, e, f, me, mf = _tc_first(x, W, asrc, adst)
        else:
            h, ta, tb, e, f, me, mf = _tc_mid(
                za, zb, aa, ab, e, f, me, mf, h, params[k - 1][3],
                W, asrc, adst)
        tap = jnp.concatenate([ta, zpadH], axis=0)
        tbp = jnp.concatenate([tb, zpadH], axis=0)
        ep = jnp.concatenate([e, zpadN])
        fp = jnp.concatenate([f, zpadN])
        me16 = jnp.broadcast_to(me.reshape(1), (16,))
        zap, zbp, aap, abp = _sc_edge(tap, tbp, ep, fp, me16, srcp, dstp)
        za = zap[:, :N]
        zb = zbp[:, :N]
        aa = aap[:, :N]
        ab = abp[:, :N]

    out = _tc_pool(za.reshape(NC, PNB, 1, PCHUNK), zb.reshape(NC, PNB, 1, PCHUNK),
                   aa, ab,
                   e.reshape(PNB, 1, PCHUNK), f.reshape(PNB, 1, PCHUNK),
                   me, mf, h, params[3][3],
                   batch.astype(jnp.int32).reshape(PNB, 1, PCHUNK))
    return out


# speedup vs baseline: 53.7658x; 1.5348x over previous
import jax
import jax.numpy as jnp
from jax import lax
from jax.experimental import pallas as pl
from jax.experimental.pallas import tpu as pltpu
from jax.experimental.pallas import tpu_sc as plsc

N = 10000
E = 320000
IN_DIM = 128
HID = 32
G = 256

NC = 2
NS = 16
NP = 10240
NPT = NP // NS
TRASH = 10200
B = 128
KB = 80
NBUF = 4
EPAD = NC * NS * KB * B
PCHUNK = 2000
PNB = N // PCHUNK


def _sc_edge_body(h_hbm, e_hbm, f_hbm, m_hbm, src_hbm, dst_hbm,
                  z_out, acc_out,
                  z_sh, acc_sh,
                  src_v, dst_v, m_v,
                  egs, fgs, ps, rowss,
                  gsems, ssems):
    cid = lax.axis_index("c")
    tid = lax.axis_index("s")
    nbase = tid * NPT

    pltpu.sync_copy(m_hbm, m_v)

    z16 = jnp.zeros((16,), jnp.float32)
    zb_v = rowss[0]
    zb1_v = ps[0]

    def zrow(r, c):
        zb_v[r, pl.ds(0, 16)] = z16
        zb_v[r, pl.ds(16, 16)] = z16
        return c
    lax.fori_loop(0, B, zrow, 0, unroll=4)

    def zrow1(k, c):
        zb1_v[pl.ds(k * 16, 16)] = z16
        return c
    lax.fori_loop(0, B // 16, zrow1, 0, unroll=4)
    for q in range(NPT // B):
        sl = pl.ds(nbase + q * B, B)
        pltpu.sync_copy(zb_v, acc_sh.at[sl])
        pltpu.sync_copy(zb1_v, z_sh.at[sl])

    g = cid * NS + tid
    pltpu.sync_copy(src_hbm.at[g], src_v)
    pltpu.sync_copy(dst_hbm.at[g], dst_v)

    plsc.subcore_barrier()
    mv = m_v[...]

    def fire_gathers(j, s):
        sidx = src_v.at[j]
        didx = dst_v.at[j]
        pltpu.async_copy(e_hbm.at[sidx], egs[s], gsems[s])
        pltpu.async_copy(f_hbm.at[didx], fgs[s], gsems[s])
        pltpu.async_copy(h_hbm.at[sidx], rowss[s], gsems[s])

    def wait_gathers(j, s):
        sidx = src_v.at[j]
        didx = dst_v.at[j]
        pltpu.make_async_copy(e_hbm.at[sidx], egs[s], gsems[s]).wait()
        pltpu.make_async_copy(f_hbm.at[didx], fgs[s], gsems[s]).wait()
        pltpu.make_async_copy(h_hbm.at[sidx], rowss[s], gsems[s]).wait()

    def compute(j, s):
        eg, fg, p_v, rows_v = egs[s], fgs[s], ps[s], rowss[s]
        for k in range(B // 16):
            sl = pl.ds(k * 16, 16)
            sv = eg[sl] + fg[sl]
            p_v[sl] = jnp.exp(jnp.maximum(sv, 0.2 * sv) - mv)

        def scale(r, c):
            pr = plsc.load_gather(p_v, [jnp.full((16,), r, jnp.int32)])
            rows_v[r, pl.ds(0, 16)] = rows_v[r, pl.ds(0, 16)] * pr
            rows_v[r, pl.ds(16, 16)] = rows_v[r, pl.ds(16, 16)] * pr
            return c
        lax.fori_loop(0, B, scale, 0, unroll=8)

    def fire_scatters(j, s):
        didx = dst_v.at[j]
        pltpu.async_copy(ps[s], z_sh.at[didx], ssems[s], add=True)
        pltpu.async_copy(rowss[s], acc_sh.at[didx], ssems[s], add=True)

    def wait_scatters(j, s):
        didx = dst_v.at[j]
        pltpu.make_async_copy(ps[s], z_sh.at[didx], ssems[s]).wait()
        pltpu.make_async_copy(rowss[s], acc_sh.at[didx], ssems[s]).wait()

    def phase(j, s):
        nxt = (s + 1) % NBUF

        @pl.when(j >= NBUF - 1)
        def _():
            wait_scatters(j, nxt)

        @pl.when(j + 1 < KB)
        def _():
            fire_gathers(j + 1, nxt)
        wait_gathers(j, s)
        compute(j, s)
        fire_scatters(j, s)

    fire_gathers(0, 0)

    def quad(i, c):
        j0 = NBUF * i
        for s in range(NBUF):
            phase(j0 + s, s)
        return c
    lax.fori_loop(0, KB // NBUF, quad, 0)
    for jt in range(KB - NBUF + 1, KB):
        wait_scatters(jt, jt % NBUF)

    plsc.subcore_barrier()
    sl = pl.ds(nbase, NPT)
    pltpu.sync_copy(z_sh.at[sl], z_out.at[cid, sl])
    pltpu.sync_copy(acc_sh.at[sl], acc_out.at[cid, sl])


def _sc_scratch():
    types = [
        pltpu.VMEM_SHARED((NP,), jnp.float32),
        pltpu.VMEM_SHARED((NP, HID), jnp.float32),
        pltpu.VMEM((KB, B), jnp.int32),
        pltpu.VMEM((KB, B), jnp.int32),
        pltpu.VMEM((16,), jnp.float32),
        [pltpu.VMEM((B,), jnp.float32) for _ in range(NBUF)],
        [pltpu.VMEM((B,), jnp.float32) for _ in range(NBUF)],
        [pltpu.VMEM((B,), jnp.float32) for _ in range(NBUF)],
        [pltpu.VMEM((B, HID), jnp.float32) for _ in range(NBUF)],
        [pltpu.SemaphoreType.DMA for _ in range(NBUF)],
        [pltpu.SemaphoreType.DMA for _ in range(NBUF)],
    ]
    return types


_sc_edge = pl.kernel(
    _sc_edge_body,
    out_type=(jax.ShapeDtypeStruct((NC, NP), jnp.float32),
              jax.ShapeDtypeStruct((NC, NP, HID), jnp.float32)),
    mesh=plsc.VectorSubcoreMesh(core_axis_name="c", subcore_axis_name="s"),
    compiler_params=pltpu.CompilerParams(needs_layout_passes=False,
                                         use_tc_tiling_on_sc=False),
    scratch_types=_sc_scratch(),
)


def _proj(h, as_ref, ad_ref):
    e = jnp.sum(h * as_ref[...][None, :], axis=1)
    f = jnp.sum(h * ad_ref[...][None, :], axis=1)
    return e, f


def _tc_first_body(x_ref, w_ref, as_ref, ad_ref, h_ref, e_ref, f_ref, m_ref):
    h = jnp.dot(x_ref[...], w_ref[...], preferred_element_type=jnp.float32)
    h_ref[...] = h
    e, f = _proj(h, as_ref, ad_ref)
    e_ref[...] = e
    f_ref[...] = f
    m_ref[0, 0] = jnp.max(e) + jnp.max(f)


_LAYER_OUT = (jax.ShapeDtypeStruct((N, HID), jnp.float32),
              jax.ShapeDtypeStruct((N,), jnp.float32),
              jax.ShapeDtypeStruct((N,), jnp.float32),
              jax.ShapeDtypeStruct((1, 1), jnp.float32))

_LAYER_OUT_SPECS = (pl.BlockSpec(memory_space=pltpu.VMEM),
                    pl.BlockSpec(memory_space=pltpu.VMEM),
                    pl.BlockSpec(memory_space=pltpu.VMEM),
                    pl.BlockSpec(memory_space=pltpu.SMEM))

_tc_first = pl.pallas_call(
    _tc_first_body,
    out_shape=_LAYER_OUT,
    out_specs=_LAYER_OUT_SPECS,
)


def _tc_mid_body(z2_ref, acc2_ref, ep_ref, fp_ref, mp_ref, hp_ref, bp_ref,
                 w_ref, as_ref, ad_ref, h_ref, e_ref, f_ref, m_ref):
    s = ep_ref[...] + fp_ref[...]
    psl = jnp.exp(jnp.maximum(s, 0.2 * s) - mp_ref[0, 0])
    z = z2_ref[0] + z2_ref[1] + psl + 1e-16
    hp = hp_ref[...]
    acc = acc2_ref[0] + acc2_ref[1] + psl[:, None] * hp
    xin = acc / z[:, None] + bp_ref[...][None, :]
    h = jnp.dot(xin, w_ref[...], preferred_element_type=jnp.float32)
    h_ref[...] = h
    e, f = _proj(h, as_ref, ad_ref)
    e_ref[...] = e
    f_ref[...] = f
    m_ref[0, 0] = jnp.max(e) + jnp.max(f)


_tc_mid = pl.pallas_call(
    _tc_mid_body,
    out_shape=_LAYER_OUT,
    out_specs=_LAYER_OUT_SPECS,
    in_specs=[pl.BlockSpec(memory_space=pltpu.VMEM),
              pl.BlockSpec(memory_space=pltpu.VMEM),
              pl.BlockSpec(memory_space=pltpu.VMEM),
              pl.BlockSpec(memory_space=pltpu.VMEM),
              pl.BlockSpec(memory_space=pltpu.SMEM),
              pl.BlockSpec(memory_space=pltpu.VMEM),
              pl.BlockSpec(memory_space=pltpu.VMEM),
              pl.BlockSpec(memory_space=pltpu.VMEM),
              pl.BlockSpec(memory_space=pltpu.VMEM),
              pl.BlockSpec(memory_space=pltpu.VMEM)],
)


def _tc_pool_body(z2_ref, acc2_ref, ep_ref, fp_ref, mp_ref, hp_ref, bp_ref,
                  batch_ref, o_ref, sums_s, cnt_s):
    i = pl.program_id(0)
    s = ep_ref[0, 0, :] + fp_ref[0, 0, :]
    psl = jnp.exp(jnp.maximum(s, 0.2 * s) - mp_ref[0, 0])
    z = z2_ref[0, 0, 0, :] + z2_ref[1, 0, 0, :] + psl + 1e-16
    hp = hp_ref[...]
    acc = acc2_ref[0] + acc2_ref[1] + psl[:, None] * hp
    xin = acc / z[:, None] + bp_ref[...][None, :]
    bt = batch_ref[0, 0, :]
    oh = (lax.broadcasted_iota(jnp.int32, (G, PCHUNK), 0)
          == bt[None, :]).astype(jnp.float32)

    @pl.when(i == 0)
    def _():
        sums_s[...] = jnp.zeros_like(sums_s)
        cnt_s[...] = jnp.zeros_like(cnt_s)

    sums_s[...] += jnp.dot(oh, xin, preferred_element_type=jnp.float32)
    cnt_s[...] += jnp.sum(oh, axis=1)

    @pl.when(i == pl.num_programs(0) - 1)
    def _():
        o_ref[...] = sums_s[...] / jnp.maximum(cnt_s[...], 1.0)[:, None]


_tc_pool = pl.pallas_call(
    _tc_pool_body,
    grid=(PNB,),
    in_specs=[
        pl.BlockSpec((NC, 1, 1, PCHUNK), lambda i: (0, i, 0, 0)),
        pl.BlockSpec((NC, PCHUNK, HID), lambda i: (0, i, 0)),
        pl.BlockSpec((1, 1, PCHUNK), lambda i: (i, 0, 0)),
        pl.BlockSpec((1, 1, PCHUNK), lambda i: (i, 0, 0)),
        pl.BlockSpec((1, 1), lambda i: (0, 0), memory_space=pltpu.SMEM),
        pl.BlockSpec((PCHUNK, HID), lambda i: (i, 0)),
        pl.BlockSpec((HID,), lambda i: (0,)),
        pl.BlockSpec((1, 1, PCHUNK), lambda i: (i, 0, 0)),
    ],
    out_specs=pl.BlockSpec((G, HID), lambda i: (0, 0)),
    out_shape=jax.ShapeDtypeStruct((G, HID), jnp.float32),
    scratch_shapes=[pltpu.VMEM((G, HID), jnp.float32),
                    pltpu.VMEM((G,), jnp.float32)],
)


def kernel(x, edge_index, batch,
           W1, a1_src, a1_dst, b1,
           W2, a2_src, a2_dst, b2,
           W3, a3_src, a3_dst, b3,
           W4, a4_src, a4_dst, b4):
    src = edge_index[0].astype(jnp.int32)
    dst = edge_index[1].astype(jnp.int32)
    pad_e = jnp.full((EPAD - E,), TRASH, jnp.int32)
    srcp = jnp.concatenate([src, pad_e]).reshape(NC * NS, KB, B)
    dstp = jnp.concatenate([dst, pad_e]).reshape(NC * NS, KB, B)
    zpadN = jnp.zeros((NP - N,), jnp.float32)
    zpadH = jnp.zeros((NP - N, HID), jnp.float32)

    params = [(W1, a1_src, a1_dst, b1), (W2, a2_src, a2_dst, b2),
              (W3, a3_src, a3_dst, b3), (W4, a4_src, a4_dst, b4)]

    h = e = f = m = z2 = acc2 = None
    for k in range(4):
        W, asrc, adst, _ = params[k]
        if k == 0:
            h, e, f, m = _tc_first(x, W, asrc, adst)
        else:
            h, e, f, m = _tc_mid(z2, acc2, e, f, m, h, params[k - 1][3],
                                 W, asrc, adst)
        hp = jnp.concatenate([h, zpadH], axis=0)
        ep = jnp.concatenate([e, zpadN])
        fp = jnp.concatenate([f, zpadN])
        m16 = jnp.broadcast_to(m.reshape(1), (16,))
        z2p, acc2p = _sc_edge(hp, ep, fp, m16, srcp, dstp)
        z2 = z2p[:, :N]
        acc2 = acc2p[:, :N]

    out = _tc_pool(z2.reshape(NC, PNB, 1, PCHUNK), acc2,
                   e.reshape(PNB, 1, PCHUNK), f.reshape(PNB, 1, PCHUNK),
                   m, h, params[3][3],
                   batch.astype(jnp.int32).reshape(PNB, 1, PCHUNK))
    return out


# speedup vs baseline: 60.7234x; 1.1294x over previous
import jax
import jax.numpy as jnp
from jax import lax
from jax.experimental import pallas as pl
from jax.experimental.pallas import tpu as pltpu
from jax.experimental.pallas import tpu_sc as plsc

N = 10000
E = 320000
IN_DIM = 128
HID = 32
G = 256

NC = 2
NS = 16
NP = 10240
NPT = NP // NS
TRASH = 10200
B = 128
KB = 80
NBUF = 4
EPAD = NC * NS * KB * B
PCHUNK = 2048
PNB = NP // PCHUNK


def _sc_edge_body(h_hbm, e_hbm, f_hbm, m_hbm, src_hbm, dst_hbm,
                  z_out, acc_out,
                  z_sh, acc_sh,
                  src_v, dst_v, m_v,
                  egs, fgs, ps, rowss,
                  gsems, ssems):
    cid = lax.axis_index("c")
    tid = lax.axis_index("s")
    nbase = tid * NPT
    psem = gsems[0]

    z16 = jnp.zeros((16,), jnp.float32)
    zb_v = rowss[0]
    zb1_v = ps[0]

    def zrow(r, c):
        zb_v[r, pl.ds(0, 16)] = z16
        zb_v[r, pl.ds(16, 16)] = z16
        return c
    lax.fori_loop(0, B, zrow, 0, unroll=4)

    def zrow1(k, c):
        zb1_v[pl.ds(k * 16, 16)] = z16
        return c
    lax.fori_loop(0, B // 16, zrow1, 0, unroll=4)

    g = cid * NS + tid
    pltpu.sync_copy(m_hbm, m_v)
    pltpu.sync_copy(src_hbm.at[g], src_v)
    pltpu.sync_copy(dst_hbm.at[g], dst_v)
    for q in range(NPT // B):
        sl = pl.ds(nbase + q * B, B)
        pltpu.sync_copy(zb_v, acc_sh.at[sl])
        pltpu.sync_copy(zb1_v, z_sh.at[sl])

    plsc.subcore_barrier()
    mv = m_v[...]

    def fire_gathers(j, s):
        sidx = src_v.at[j]
        didx = dst_v.at[j]
        pltpu.async_copy(e_hbm.at[sidx], egs[s], gsems[s])
        pltpu.async_copy(f_hbm.at[didx], fgs[s], gsems[s])
        pltpu.async_copy(h_hbm.at[sidx], rowss[s], gsems[s])

    def wait_gathers(j, s):
        sidx = src_v.at[j]
        didx = dst_v.at[j]
        pltpu.make_async_copy(e_hbm.at[sidx], egs[s], gsems[s]).wait()
        pltpu.make_async_copy(f_hbm.at[didx], fgs[s], gsems[s]).wait()
        pltpu.make_async_copy(h_hbm.at[sidx], rowss[s], gsems[s]).wait()

    def compute(j, s):
        eg, fg, p_v, rows_v = egs[s], fgs[s], ps[s], rowss[s]
        for k in range(B // 16):
            sl = pl.ds(k * 16, 16)
            sv = eg[sl] + fg[sl]
            p_v[sl] = jnp.exp(jnp.maximum(sv, 0.2 * sv) - mv)

        def scale(r, c):
            pr = plsc.load_gather(p_v, [jnp.full((16,), r, jnp.int32)])
            rows_v[r, pl.ds(0, 16)] = rows_v[r, pl.ds(0, 16)] * pr
            rows_v[r, pl.ds(16, 16)] = rows_v[r, pl.ds(16, 16)] * pr
            return c
        lax.fori_loop(0, B, scale, 0, unroll=8)

    def fire_scatters(j, s):
        didx = dst_v.at[j]
        pltpu.async_copy(ps[s], z_sh.at[didx], ssems[s], add=True)
        pltpu.async_copy(rowss[s], acc_sh.at[didx], ssems[s], add=True)

    def wait_scatters(j, s):
        didx = dst_v.at[j]
        pltpu.make_async_copy(ps[s], z_sh.at[didx], ssems[s]).wait()
        pltpu.make_async_copy(rowss[s], acc_sh.at[didx], ssems[s]).wait()

    def phase(j, s):
        nxt = (s + 1) % NBUF

        @pl.when(j >= NBUF - 1)
        def _():
            wait_scatters(j, nxt)

        @pl.when(j + 1 < KB)
        def _():
            fire_gathers(j + 1, nxt)
        wait_gathers(j, s)
        compute(j, s)
        fire_scatters(j, s)

    fire_gathers(0, 0)

    def quad(i, c):
        j0 = NBUF * i
        for s in range(NBUF):
            phase(j0 + s, s)
        return c
    lax.fori_loop(0, KB // NBUF, quad, 0)
    for jt in range(KB - NBUF + 1, KB):
        wait_scatters(jt, jt % NBUF)

    plsc.subcore_barrier()
    sl = pl.ds(nbase, NPT)
    pltpu.async_copy(z_sh.at[sl], z_out.at[cid, sl], psem)
    pltpu.async_copy(acc_sh.at[sl], acc_out.at[cid, sl], psem)
    pltpu.make_async_copy(z_sh.at[sl], z_out.at[cid, sl], psem).wait()
    pltpu.make_async_copy(acc_sh.at[sl], acc_out.at[cid, sl], psem).wait()


def _sc_scratch():
    return [
        pltpu.VMEM_SHARED((NP,), jnp.float32),
        pltpu.VMEM_SHARED((NP, HID), jnp.float32),
        pltpu.VMEM((KB, B), jnp.int32),
        pltpu.VMEM((KB, B), jnp.int32),
        pltpu.VMEM((16,), jnp.float32),
        [pltpu.VMEM((B,), jnp.float32) for _ in range(NBUF)],
        [pltpu.VMEM((B,), jnp.float32) for _ in range(NBUF)],
        [pltpu.VMEM((B,), jnp.float32) for _ in range(NBUF)],
        [pltpu.VMEM((B, HID), jnp.float32) for _ in range(NBUF)],
        [pltpu.SemaphoreType.DMA for _ in range(NBUF)],
        [pltpu.SemaphoreType.DMA for _ in range(NBUF)],
    ]


_sc_edge = pl.kernel(
    _sc_edge_body,
    out_type=(jax.ShapeDtypeStruct((NC, NP), jnp.float32),
              jax.ShapeDtypeStruct((NC, NP, HID), jnp.float32)),
    mesh=plsc.VectorSubcoreMesh(core_axis_name="c", subcore_axis_name="s"),
    compiler_params=pltpu.CompilerParams(needs_layout_passes=False,
                                         use_tc_tiling_on_sc=False),
    scratch_types=_sc_scratch(),
)


def _proj(h, as_ref, ad_ref):
    e = jnp.sum(h * as_ref[...][None, :], axis=1)
    f = jnp.sum(h * ad_ref[...][None, :], axis=1)
    return e, f


def _tc_first_body(x_ref, w_ref, as_ref, ad_ref, h_ref, e_ref, f_ref, m_ref):
    h = jnp.dot(x_ref[...], w_ref[...], preferred_element_type=jnp.float32)
    h_ref[...] = h
    e, f = _proj(h, as_ref, ad_ref)
    e_ref[...] = e
    f_ref[...] = f
    m_ref[0, 0] = jnp.max(e) + jnp.max(f)


_LAYER_OUT = (jax.ShapeDtypeStruct((NP, HID), jnp.float32),
              jax.ShapeDtypeStruct((NP,), jnp.float32),
              jax.ShapeDtypeStruct((NP,), jnp.float32),
              jax.ShapeDtypeStruct((1, 1), jnp.float32))

_LAYER_OUT_SPECS = (pl.BlockSpec(memory_space=pltpu.VMEM),
                    pl.BlockSpec(memory_space=pltpu.VMEM),
                    pl.BlockSpec(memory_space=pltpu.VMEM),
                    pl.BlockSpec(memory_space=pltpu.SMEM))

_tc_first = pl.pallas_call(
    _tc_first_body,
    out_shape=_LAYER_OUT,
    out_specs=_LAYER_OUT_SPECS,
)


def _tc_mid_body(z2_ref, acc2_ref, ep_ref, fp_ref, mp_ref, hp_ref, bp_ref,
                 w_ref, as_ref, ad_ref, h_ref, e_ref, f_ref, m_ref):
    s = ep_ref[...] + fp_ref[...]
    psl = jnp.exp(jnp.maximum(s, 0.2 * s) - mp_ref[0, 0])
    z = z2_ref[0] + z2_ref[1] + psl + 1e-16
    hp = hp_ref[...]
    acc = acc2_ref[0] + acc2_ref[1] + psl[:, None] * hp
    xin = acc / z[:, None] + bp_ref[...][None, :]
    h = jnp.dot(xin, w_ref[...], preferred_element_type=jnp.float32)
    h_ref[...] = h
    e, f = _proj(h, as_ref, ad_ref)
    e_ref[...] = e
    f_ref[...] = f
    m_ref[0, 0] = jnp.max(e) + jnp.max(f)


_tc_mid = pl.pallas_call(
    _tc_mid_body,
    out_shape=_LAYER_OUT,
    out_specs=_LAYER_OUT_SPECS,
    in_specs=[pl.BlockSpec(memory_space=pltpu.VMEM),
              pl.BlockSpec(memory_space=pltpu.VMEM),
              pl.BlockSpec(memory_space=pltpu.VMEM),
              pl.BlockSpec(memory_space=pltpu.VMEM),
              pl.BlockSpec(memory_space=pltpu.SMEM),
              pl.BlockSpec(memory_space=pltpu.VMEM),
              pl.BlockSpec(memory_space=pltpu.VMEM),
              pl.BlockSpec(memory_space=pltpu.VMEM),
              pl.BlockSpec(memory_space=pltpu.VMEM),
              pl.BlockSpec(memory_space=pltpu.VMEM)],
)


def _tc_pool_body(z2_ref, acc2_ref, ep_ref, fp_ref, mp_ref, hp_ref, bp_ref,
                  batch_ref, o_ref, sums_s, cnt_s):
    i = pl.program_id(0)
    s = ep_ref[0, 0, :] + fp_ref[0, 0, :]
    psl = jnp.exp(jnp.maximum(s, 0.2 * s) - mp_ref[0, 0])
    z = z2_ref[0, 0, 0, :] + z2_ref[1, 0, 0, :] + psl + 1e-16
    hp = hp_ref[...]
    acc = acc2_ref[0] + acc2_ref[1] + psl[:, None] * hp
    xin = acc / z[:, None] + bp_ref[...][None, :]
    bt = batch_ref[0, 0, :]
    oh = (lax.broadcasted_iota(jnp.int32, (G, PCHUNK), 0)
          == bt[None, :]).astype(jnp.float32)

    @pl.when(i == 0)
    def _():
        sums_s[...] = jnp.zeros_like(sums_s)
        cnt_s[...] = jnp.zeros_like(cnt_s)

    sums_s[...] += jnp.dot(oh, xin, preferred_element_type=jnp.float32)
    cnt_s[...] += jnp.sum(oh, axis=1)

    @pl.when(i == pl.num_programs(0) - 1)
    def _():
        o_ref[...] = sums_s[...] / jnp.maximum(cnt_s[...], 1.0)[:, None]


_tc_pool = pl.pallas_call(
    _tc_pool_body,
    grid=(PNB,),
    in_specs=[
        pl.BlockSpec((NC, 1, 1, PCHUNK), lambda i: (0, i, 0, 0)),
        pl.BlockSpec((NC, PCHUNK, HID), lambda i: (0, i, 0)),
        pl.BlockSpec((1, 1, PCHUNK), lambda i: (i, 0, 0)),
        pl.BlockSpec((1, 1, PCHUNK), lambda i: (i, 0, 0)),
        pl.BlockSpec((1, 1), lambda i: (0, 0), memory_space=pltpu.SMEM),
        pl.BlockSpec((PCHUNK, HID), lambda i: (i, 0)),
        pl.BlockSpec((HID,), lambda i: (0,)),
        pl.BlockSpec((1, 1, PCHUNK), lambda i: (i, 0, 0)),
    ],
    out_specs=pl.BlockSpec((G, HID), lambda i: (0, 0)),
    out_shape=jax.ShapeDtypeStruct((G, HID), jnp.float32),
    scratch_shapes=[pltpu.VMEM((G, HID), jnp.float32),
                    pltpu.VMEM((G,), jnp.float32)],
)


def kernel(x, edge_index, batch,
           W1, a1_src, a1_dst, b1,
           W2, a2_src, a2_dst, b2,
           W3, a3_src, a3_dst, b3,
           W4, a4_src, a4_dst, b4):
    src = edge_index[0].astype(jnp.int32)
    dst = edge_index[1].astype(jnp.int32)
    pad_e = jnp.full((EPAD - E,), TRASH, jnp.int32)
    srcp = jnp.concatenate([src, pad_e]).reshape(NC * NS, KB, B)
    dstp = jnp.concatenate([dst, pad_e]).reshape(NC * NS, KB, B)
    xp = jnp.concatenate(
        [x, jnp.zeros((NP - N, IN_DIM), jnp.float32)], axis=0)
    batchp = jnp.concatenate(
        [batch.astype(jnp.int32), jnp.full((NP - N,), G, jnp.int32)])

    params = [(W1, a1_src, a1_dst, b1), (W2, a2_src, a2_dst, b2),
              (W3, a3_src, a3_dst, b3), (W4, a4_src, a4_dst, b4)]

    h = e = f = m = z2 = acc2 = None
    for k in range(4):
        W, asrc, adst, _ = params[k]
        if k == 0:
            h, e, f, m = _tc_first(xp, W, asrc, adst)
        else:
            h, e, f, m = _tc_mid(z2, acc2, e, f, m, h, params[k - 1][3],
                                 W, asrc, adst)
        m16 = jnp.broadcast_to(m.reshape(1), (16,))
        z2, acc2 = _sc_edge(h, e, f, m16, srcp, dstp)

    out = _tc_pool(z2.reshape(NC, PNB, 1, PCHUNK), acc2,
                   e.reshape(PNB, 1, PCHUNK), f.reshape(PNB, 1, PCHUNK),
                   m, h, params[3][3],
                   batchp.reshape(PNB, 1, PCHUNK))
    return out
